# baseline (speedup 1.0000x reference)
import jax
import jax.numpy as jnp
from jax import lax
from jax.experimental import pallas as pl
from jax.experimental.pallas import tpu as pltpu, tpu_sc as plsc

N_NODES = 10000
N_EDGES = 320000
D = 128
DW = 144
NEG_SLOPE = 0.2

NC = 2
NS = 16
NW = NC * NS
ACC_ROWS = 10240
STRIPE = ACC_ROWS // NW
DC = D // 16
E_PAD = 327680
GR = 8
NGR = E_PAD // (GR * 128)
FB = 32
CAP = 13568
CBUF = CAP + 128


def _proj_body(x_ref, wl_ref, wr_ref, hl_ref, hr_ref):
    x = x_ref[...]
    hl_ref[...] = jnp.dot(x, wl_ref[...], preferred_element_type=jnp.float32)
    hr_ref[...] = jnp.dot(x, wr_ref[...], preferred_element_type=jnp.float32)


def _project(x, W_l, W_r):
    blk = 2000
    grid = (N_NODES // blk,)
    return pl.pallas_call(
        _proj_body,
        grid=grid,
        in_specs=[
            pl.BlockSpec((blk, D), lambda i: (i, 0)),
            pl.BlockSpec((D, D), lambda i: (0, 0)),
            pl.BlockSpec((D, D), lambda i: (0, 0)),
        ],
        out_specs=[
            pl.BlockSpec((blk, D), lambda i: (i, 0)),
            pl.BlockSpec((blk, D), lambda i: (i, 0)),
        ],
        out_shape=[
            jax.ShapeDtypeStruct((N_NODES, D), jnp.float32),
            jax.ShapeDtypeStruct((N_NODES, D), jnp.float32),
        ],
    )(x, W_l, W_r)


def _sc_edge_body(hl_hbm, hr_hbm, src_hbm, dst_hbm, a_hbm,
                  out_hbm,
                  sg, dg, a_v, csrc, cdst, Ws, Wd, A0, R0, A1, R1,
                  semA0, semR0, semA1, semR1, acc):
    cid = lax.axis_index("c")
    sid = lax.axis_index("s")
    wid = sid * NC + cid
    lo = wid * STRIPE
    hi = lo + STRIPE

    pltpu.sync_copy(a_hbm, a_v)

    zero16 = jnp.zeros((16,), jnp.float32)
    lane = lax.iota(jnp.int32, 16)
    onehot = jnp.where(lane == 0, 1.0, 0.0).astype(jnp.float32)

    def zrow(r, c0):
        for c in range(DW // 16):
            acc[r, pl.ds(c * 16, 16)] = zero16
        return c0
    lax.fori_loop(0, STRIPE + 8, zrow, 0)

    def start(b, A_, R_, semA, semR):
        base = pl.multiple_of(b * FB, FB)
        pltpu.async_copy(hl_hbm.at[cdst.at[pl.ds(base, FB)]], A_, semA)
        pltpu.async_copy(hr_hbm.at[csrc.at[pl.ds(base, FB)]], R_, semR)

    def wait(A_, R_, semA, semR):
        pltpu.make_async_copy(hl_hbm.at[pl.ds(0, FB)], A_, semA).wait()
        pltpu.make_async_copy(hr_hbm.at[pl.ds(0, FB)], R_, semR).wait()

    def compute(b, A, R):
        base = pl.multiple_of(b * FB, FB)

        def grp_body(g, c1):
            off = pl.multiple_of(base + g * 16, 16)
            d16 = cdst[pl.ds(off, 16)] - jnp.full((16,), lo, jnp.int32)
            for i in range(16):
                r = g * 16 + i
                d = d16[i]
                acc16 = zero16
                for c in range(DC):
                    t = A[r, pl.ds(c * 16, 16)] + R[r, pl.ds(c * 16, 16)]
                    t = jnp.maximum(t, t * NEG_SLOPE)
                    acc16 = acc16 + a_v[c] * t
                e = jnp.sum(acc16)
                w = jnp.exp(jnp.full((16,), e, jnp.float32))
                for c in range(DC):
                    acc[d, pl.ds(c * 16, 16)] = (
                        acc[d, pl.ds(c * 16, 16)] + R[r, pl.ds(c * 16, 16)] * w)
                acc[d, pl.ds(D, 16)] = acc[d, pl.ds(D, 16)] + w * onehot
            return c1
        lax.fori_loop(0, FB // 16, grp_body, 0)

    def group_body(g, cnt):
        pltpu.sync_copy(src_hbm.at[pl.ds(g * GR, GR)], sg)
        pltpu.sync_copy(dst_hbm.at[pl.ds(g * GR, GR)], dg)

        def qstep(q, cnt1):
            for c in range(8):
                vd = dg[q, pl.ds(c * 16, 16)]
                vs = sg[q, pl.ds(c * 16, 16)]
                m = (vd >= lo) & (vd < hi)
                plsc.store_compressed(Wd.at[pl.ds(0, 16)], vd, mask=m)
                plsc.store_compressed(Ws.at[pl.ds(0, 16)], vs, mask=m)
                k = plsc.all_reduce_population_count(m)[0]
                pos = jnp.full((16,), cnt1, jnp.int32) + lane
                km = (lane < jnp.full((16,), k, jnp.int32)) & (
                    pos < jnp.full((16,), CAP, jnp.int32))
                plsc.store_scatter(cdst, [pos], Wd[pl.ds(0, 16)], mask=km)
                plsc.store_scatter(csrc, [pos], Ws[pl.ds(0, 16)], mask=km)
                cnt1 = cnt1 + k
            return cnt1
        return lax.fori_loop(0, GR, qstep, cnt)
    cnt = lax.fori_loop(0, NGR, group_body, 0)
    cnt = jnp.minimum(cnt, CAP)

    hi16 = jnp.full((16,), hi, jnp.int32)
    zi16 = jnp.zeros((16,), jnp.int32)
    cnt16 = jnp.full((16,), cnt, jnp.int32)
    for k in range(8):
        pos = cnt16 + lane + jnp.full((16,), k * 16, jnp.int32)
        plsc.store_scatter(cdst, [pos], hi16)
        plsc.store_scatter(csrc, [pos], zi16)

    nb2 = jnp.maximum((cnt + 2 * FB - 1) // (2 * FB), 1)
    start(0, A0, R0, semA0, semR0)

    def pair_body(i2, c0):
        b0 = 2 * i2
        wait(A0, R0, semA0, semR0)
        start(b0 + 1, A1, R1, semA1, semR1)
        compute(b0, A0, R0)
        wait(A1, R1, semA1, semR1)

        @pl.when(i2 + 1 < nb2)
        def _():
            start(b0 + 2, A0, R0, semA0, semR0)
        compute(b0 + 1, A1, R1)
        return c0
    lax.fori_loop(0, nb2, pair_body, 0)

    pltpu.sync_copy(acc.at[pl.ds(0, STRIPE)], out_hbm.at[pl.ds(lo, STRIPE)])


def _sc_edges(h_l, h_r, src, dst, a2d):
    mesh = plsc.VectorSubcoreMesh(core_axis_name="c", subcore_axis_name="s")
    return pl.kernel(
        _sc_edge_body,
        out_type=jax.ShapeDtypeStruct((ACC_ROWS, DW), jnp.float32),
        mesh=mesh,
        compiler_params=pltpu.CompilerParams(needs_layout_passes=False),
        scratch_types=[
            pltpu.VMEM((GR, 128), jnp.int32),
            pltpu.VMEM((GR, 128), jnp.int32),
            pltpu.VMEM((DC, 16), jnp.float32),
            pltpu.VMEM((CBUF,), jnp.int32),
            pltpu.VMEM((CBUF,), jnp.int32),
            pltpu.VMEM((16,), jnp.int32),
            pltpu.VMEM((16,), jnp.int32),
            pltpu.VMEM((FB, D), jnp.float32),
            pltpu.VMEM((FB, D), jnp.float32),
            pltpu.VMEM((FB, D), jnp.float32),
            pltpu.VMEM((FB, D), jnp.float32),
            pltpu.SemaphoreType.DMA,
            pltpu.SemaphoreType.DMA,
            pltpu.SemaphoreType.DMA,
            pltpu.SemaphoreType.DMA,
            pltpu.VMEM((STRIPE + 8, DW), jnp.float32),
        ],
    )(h_l, h_r, src, dst, a2d)


def _fin_body(md_ref, bias_ref, out_ref):
    m = md_ref[:, 0:D]
    d = md_ref[:, D:D + 1]
    out_ref[...] = m / (d + 1e-9) + bias_ref[...]


def _finish(md, bias2d):
    blk = 1000
    grid = (N_NODES // blk,)
    return pl.pallas_call(
        _fin_body,
        grid=grid,
        in_specs=[
            pl.BlockSpec((blk, DW), lambda i: (i, 0)),
            pl.BlockSpec((1, D), lambda i: (0, 0)),
        ],
        out_specs=pl.BlockSpec((blk, D), lambda i: (i, 0)),
        out_shape=jax.ShapeDtypeStruct((N_NODES, D), jnp.float32),
    )(md, bias2d)


def kernel(inst_feat, edge_index, W_l, W_r, a, bias):
    pad = E_PAD - N_EDGES
    src = jnp.concatenate(
        [edge_index[0].astype(jnp.int32), jnp.zeros((pad,), jnp.int32)]
    ).reshape(GR * NGR, 128)
    dst = jnp.concatenate(
        [edge_index[1].astype(jnp.int32),
         jnp.full((pad,), jnp.int32(1 << 29))]
    ).reshape(GR * NGR, 128)
    a2d = a.astype(jnp.float32).reshape(DC, 16)
    bias2d = bias.astype(jnp.float32).reshape(1, D)

    h_l, h_r = _project(inst_feat, W_l, W_r)
    md = _sc_edges(h_l, h_r, src, dst, a2d)
    return _finish(md, bias2d)


# speedup vs baseline: 1.5297x; 1.5297x over previous
import jax
import jax.numpy as jnp
from jax import lax
from jax.experimental import pallas as pl
from jax.experimental.pallas import tpu as pltpu, tpu_sc as plsc

N_NODES = 10000
N_EDGES = 320000
D = 128
DW = 144
NEG_SLOPE = 0.2

NC = 2
NS = 16
NW = NC * NS
ACC_ROWS = 10240
STRIPE = ACC_ROWS // NW
DC = D // 16
E_PAD = 327680
GR = 8
NGR = E_PAD // (GR * 128)
FB = 32
CAP = 13568
CBUF = CAP + 128


def _proj_body(x_ref, wl_ref, wr_ref, hl_ref, hr_ref):
    x = x_ref[...]
    hl_ref[...] = jnp.dot(x, wl_ref[...], preferred_element_type=jnp.float32)
    hr_ref[...] = jnp.dot(x, wr_ref[...], preferred_element_type=jnp.float32)


def _project(x, W_l, W_r):
    blk = 2000
    grid = (N_NODES // blk,)
    return pl.pallas_call(
        _proj_body,
        grid=grid,
        in_specs=[
            pl.BlockSpec((blk, D), lambda i: (i, 0)),
            pl.BlockSpec((D, D), lambda i: (0, 0)),
            pl.BlockSpec((D, D), lambda i: (0, 0)),
        ],
        out_specs=[
            pl.BlockSpec((blk, D), lambda i: (i, 0)),
            pl.BlockSpec((blk, D), lambda i: (i, 0)),
        ],
        out_shape=[
            jax.ShapeDtypeStruct((N_NODES, D), jnp.float32),
            jax.ShapeDtypeStruct((N_NODES, D), jnp.float32),
        ],
    )(x, W_l, W_r)


def _sc_edge_body(hl_hbm, hr_hbm, src_hbm, dst_hbm, a_hbm,
                  out_hbm,
                  sg, dg, a_v, csrc, cdst, Ws, Wd, A0, R0, A1, R1,
                  semA0, semR0, semA1, semR1, acc):
    cid = lax.axis_index("c")
    sid = lax.axis_index("s")
    wid = sid * NC + cid
    lo = wid * STRIPE
    hi = lo + STRIPE

    pltpu.sync_copy(a_hbm, a_v)

    zero16 = jnp.zeros((16,), jnp.float32)
    lane = lax.iota(jnp.int32, 16)
    onehot = jnp.where(lane == 0, 1.0, 0.0).astype(jnp.float32)

    def zrow(r, c0):
        for c in range(DW // 16):
            acc[r, pl.ds(c * 16, 16)] = zero16
        return c0
    lax.fori_loop(0, STRIPE + 8, zrow, 0)

    def start(b, A_, R_, semA, semR):
        base = pl.multiple_of(b * FB, FB)
        pltpu.async_copy(hl_hbm.at[cdst.at[pl.ds(base, FB)]], A_, semA)
        pltpu.async_copy(hr_hbm.at[csrc.at[pl.ds(base, FB)]], R_, semR)

    def wait(A_, R_, semA, semR):
        pltpu.make_async_copy(hl_hbm.at[pl.ds(0, FB)], A_, semA).wait()
        pltpu.make_async_copy(hr_hbm.at[pl.ds(0, FB)], R_, semR).wait()

    def compute(b, A, R):
        base = pl.multiple_of(b * FB, FB)

        def row_body(r, c1):
            d16 = plsc.load_gather(cdst, [jnp.full((16,), base + r, jnp.int32)])
            d = d16[0] - lo
            acc16 = zero16
            for c in range(DC):
                t = A[r, pl.ds(c * 16, 16)] + R[r, pl.ds(c * 16, 16)]
                t = jnp.maximum(t, t * NEG_SLOPE)
                acc16 = acc16 + a_v[c] * t
            e = jnp.sum(acc16)
            w = jnp.exp(jnp.full((16,), e, jnp.float32))
            for c in range(DC):
                acc[d, pl.ds(c * 16, 16)] = (
                    acc[d, pl.ds(c * 16, 16)] + R[r, pl.ds(c * 16, 16)] * w)
            acc[d, pl.ds(D, 16)] = acc[d, pl.ds(D, 16)] + w * onehot
            return c1
        lax.fori_loop(0, FB, row_body, 0)

    def group_body(g, cnt):
        pltpu.sync_copy(src_hbm.at[pl.ds(g * GR, GR)], sg)
        pltpu.sync_copy(dst_hbm.at[pl.ds(g * GR, GR)], dg)

        def qstep(q, cnt1):
            for c in range(8):
                vd = dg[q, pl.ds(c * 16, 16)]
                vs = sg[q, pl.ds(c * 16, 16)]
                m = (vd >= lo) & (vd < hi)
                plsc.store_compressed(Wd.at[pl.ds(0, 16)], vd, mask=m)
                plsc.store_compressed(Ws.at[pl.ds(0, 16)], vs, mask=m)
                k = plsc.all_reduce_population_count(m)[0]
                pos = jnp.full((16,), cnt1, jnp.int32) + lane
                km = (lane < jnp.full((16,), k, jnp.int32)) & (
                    pos < jnp.full((16,), CAP, jnp.int32))
                plsc.store_scatter(cdst, [pos], Wd[pl.ds(0, 16)], mask=km)
                plsc.store_scatter(csrc, [pos], Ws[pl.ds(0, 16)], mask=km)
                cnt1 = cnt1 + k
            return cnt1
        return lax.fori_loop(0, GR, qstep, cnt)
    cnt = lax.fori_loop(0, NGR, group_body, 0)
    cnt = jnp.minimum(cnt, CAP)

    hi16 = jnp.full((16,), hi, jnp.int32)
    zi16 = jnp.zeros((16,), jnp.int32)
    cnt16 = jnp.full((16,), cnt, jnp.int32)
    for k in range(8):
        pos = cnt16 + lane + jnp.full((16,), k * 16, jnp.int32)
        plsc.store_scatter(cdst, [pos], hi16)
        plsc.store_scatter(csrc, [pos], zi16)

    nb2 = jnp.maximum((cnt + 2 * FB - 1) // (2 * FB), 1)
    start(0, A0, R0, semA0, semR0)

    def pair_body(i2, c0):
        b0 = 2 * i2
        wait(A0, R0, semA0, semR0)
        start(b0 + 1, A1, R1, semA1, semR1)
        compute(b0, A0, R0)
        wait(A1, R1, semA1, semR1)

        @pl.when(i2 + 1 < nb2)
        def _():
            start(b0 + 2, A0, R0, semA0, semR0)
        compute(b0 + 1, A1, R1)
        return c0
    lax.fori_loop(0, nb2, pair_body, 0)

    pltpu.sync_copy(acc.at[pl.ds(0, STRIPE)], out_hbm.at[pl.ds(lo, STRIPE)])


def _sc_edges(h_l, h_r, src, dst, a2d):
    mesh = plsc.VectorSubcoreMesh(core_axis_name="c", subcore_axis_name="s")
    return pl.kernel(
        _sc_edge_body,
        out_type=jax.ShapeDtypeStruct((ACC_ROWS, DW), jnp.float32),
        mesh=mesh,
        compiler_params=pltpu.CompilerParams(needs_layout_passes=False),
        scratch_types=[
            pltpu.VMEM((GR, 128), jnp.int32),
            pltpu.VMEM((GR, 128), jnp.int32),
            pltpu.VMEM((DC, 16), jnp.float32),
            pltpu.VMEM((CBUF,), jnp.int32),
            pltpu.VMEM((CBUF,), jnp.int32),
            pltpu.VMEM((16,), jnp.int32),
            pltpu.VMEM((16,), jnp.int32),
            pltpu.VMEM((FB, D), jnp.float32),
            pltpu.VMEM((FB, D), jnp.float32),
            pltpu.VMEM((FB, D), jnp.float32),
            pltpu.VMEM((FB, D), jnp.float32),
            pltpu.SemaphoreType.DMA,
            pltpu.SemaphoreType.DMA,
            pltpu.SemaphoreType.DMA,
            pltpu.SemaphoreType.DMA,
            pltpu.VMEM((STRIPE + 8, DW), jnp.float32),
        ],
    )(h_l, h_r, src, dst, a2d)


def _fin_body(md_ref, bias_ref, out_ref):
    m = md_ref[:, 0:D]
    d = md_ref[:, D:D + 1]
    out_ref[...] = m / (d + 1e-9) + bias_ref[...]


def _finish(md, bias2d):
    blk = 1000
    grid = (N_NODES // blk,)
    return pl.pallas_call(
        _fin_body,
        grid=grid,
        in_specs=[
            pl.BlockSpec((blk, DW), lambda i: (i, 0)),
            pl.BlockSpec((1, D), lambda i: (0, 0)),
        ],
        out_specs=pl.BlockSpec((blk, D), lambda i: (i, 0)),
        out_shape=jax.ShapeDtypeStruct((N_NODES, D), jnp.float32),
    )(md, bias2d)


def kernel(inst_feat, edge_index, W_l, W_r, a, bias):
    pad = E_PAD - N_EDGES
    src = jnp.concatenate(
        [edge_index[0].astype(jnp.int32), jnp.zeros((pad,), jnp.int32)]
    ).reshape(GR * NGR, 128)
    dst = jnp.concatenate(
        [edge_index[1].astype(jnp.int32),
         jnp.full((pad,), jnp.int32(1 << 29))]
    ).reshape(GR * NGR, 128)
    a2d = a.astype(jnp.float32).reshape(DC, 16)
    bias2d = bias.astype(jnp.float32).reshape(1, D)

    h_l, h_r = _project(inst_feat, W_l, W_r)
    md = _sc_edges(h_l, h_r, src, dst, a2d)
    return _finish(md, bias2d)
